# Initial kernel scaffold; baseline (speedup 1.0000x reference)
import functools

import jax
import jax.numpy as jnp
from jax import lax
from jax.experimental import pallas as pl
from jax.experimental.pallas import tpu as pltpu
from jax.experimental.pallas import tpu_sc as plsc

N = 10000
E = 320000
HID = 128
G = 64

NC = 2
NS = 16
NW = NC * NS
EPW = E // NW
CH = 80
NCHUNK = EPW // CH
RPW = N // NS
ZR = 25

_MESH = plsc.VectorSubcoreMesh(core_axis_name="c", subcore_axis_name="s")
_HIGH = jax.lax.Precision.HIGHEST


@functools.partial(
    pl.kernel,
    mesh=_MESH,
    out_type=jax.ShapeDtypeStruct((NC, N, 16), jnp.float32),
    scratch_types=[
        pltpu.VMEM((CH,), jnp.int32),
        pltpu.VMEM((CH, 16), jnp.float32),
        pltpu.VMEM((ZR, 16), jnp.float32),
        pltpu.VMEM_SHARED((N, 16), jnp.float32),
    ],
)
def _sc_count(dst_hbm, out_hbm, didx, ones_v, zbuf, acc_sh):
    c = lax.axis_index("c")
    s = lax.axis_index("s")

    @pl.loop(0, CH)
    def _(i):
        ones_v[i, :] = jnp.ones((16,), jnp.float32)

    @pl.loop(0, ZR)
    def _(i):
        zbuf[i, :] = jnp.zeros((16,), jnp.float32)

    @pl.loop(0, RPW // ZR)
    def _(i):
        pltpu.sync_copy(zbuf, acc_sh.at[pl.ds(s * RPW + i * ZR, ZR)])

    plsc.subcore_barrier()

    base = (c * NS + s) * EPW

    @pl.loop(0, NCHUNK)
    def _(k):
        pltpu.sync_copy(dst_hbm.at[pl.ds(base + k * CH, CH)], didx)
        pltpu.sync_copy(ones_v, acc_sh.at[didx], add=True)

    plsc.subcore_barrier()
    pltpu.sync_copy(acc_sh.at[pl.ds(s * RPW, RPW)],
                    out_hbm.at[c].at[pl.ds(s * RPW, RPW)])


@functools.partial(
    pl.kernel,
    mesh=_MESH,
    out_type=jax.ShapeDtypeStruct((NC, N, HID), jnp.float32),
    scratch_types=[
        pltpu.VMEM((CH,), jnp.int32),
        pltpu.VMEM((CH,), jnp.int32),
        pltpu.VMEM((CH, HID), jnp.float32),
        pltpu.VMEM((ZR, HID), jnp.float32),
        pltpu.VMEM_SHARED((N, HID), jnp.float32),
        pltpu.SemaphoreType.DMA,
    ],
)
def _sc_scatter(src_hbm, dst_hbm, h_hbm, out_hbm, sidx, didx, rows, zbuf,
                acc_sh, sem):
    c = lax.axis_index("c")
    s = lax.axis_index("s")

    @pl.loop(0, ZR)
    def _(i):
        @pl.loop(0, HID // 16)
        def _(j):
            zbuf[i, pl.ds(j * 16, 16)] = jnp.zeros((16,), jnp.float32)

    @pl.loop(0, RPW // ZR)
    def _(i):
        pltpu.sync_copy(zbuf, acc_sh.at[pl.ds(s * RPW + i * ZR, ZR)])

    plsc.subcore_barrier()

    base = (c * NS + s) * EPW

    @pl.loop(0, NCHUNK)
    def _(k):
        pltpu.sync_copy(src_hbm.at[pl.ds(base + k * CH, CH)], sidx)
        pltpu.sync_copy(dst_hbm.at[pl.ds(base + k * CH, CH)], didx)
        pltpu.async_copy(h_hbm.at[sidx], rows, sem).wait()
        pltpu.sync_copy(rows, acc_sh.at[didx], add=True)

    plsc.subcore_barrier()
    pltpu.sync_copy(acc_sh.at[pl.ds(s * RPW, RPW)],
                    out_hbm.at[c].at[pl.ds(s * RPW, RPW)])


def _matmul_t(a, b):
    return lax.dot_general(a, b, (((1,), (1,)), ((), ())),
                           preferred_element_type=jnp.float32,
                           precision=_HIGH)


def _tc_pre_body(x_ref, w1_ref, h1_ref):
    h1_ref[...] = _matmul_t(x_ref[...], w1_ref[...])


def _tc_mid1_body(cnt_ref, h1_ref, hs1_ref, dv_ref):
    deg = cnt_ref[0, :, 0:1] + cnt_ref[1, :, 0:1] + 1.0
    dv = lax.rsqrt(deg)
    dv_ref[...] = dv
    hs1_ref[...] = h1_ref[...] * dv


def _bn_relu(agg, g_ref, be_ref):
    mu = jnp.mean(agg, axis=0, keepdims=True)
    var = jnp.mean((agg - mu) ** 2, axis=0, keepdims=True)
    return jax.nn.relu(g_ref[...] * (agg - mu) * lax.rsqrt(var + 1e-5)
                       + be_ref[...])


def _tc_mid2_body(p_ref, h1_ref, dv_ref, b1_ref, g1_ref, be1_ref, w2_ref,
                  h2_ref, hs2_ref):
    dv = dv_ref[...]
    agg = (p_ref[0] + p_ref[1]) * dv + dv * dv * h1_ref[...] + b1_ref[...]
    y = _bn_relu(agg, g1_ref, be1_ref)
    h2 = _matmul_t(y, w2_ref[...])
    h2_ref[...] = h2
    hs2_ref[...] = h2 * dv


def _tc_final_body(p_ref, h2_ref, dv_ref, b2_ref, g2_ref, be2_ref, batch_ref,
                   wl1_ref, bl1_ref, wl2_ref, bl2_ref, out_ref):
    dv = dv_ref[...]
    agg = (p_ref[0] + p_ref[1]) * dv + dv * dv * h2_ref[...] + b2_ref[...]
    y = _bn_relu(agg, g2_ref, be2_ref)
    onehot = (batch_ref[...] == lax.broadcasted_iota(jnp.int32, (G, N), 0)
              ).astype(jnp.float32)
    sums = lax.dot_general(onehot, y, (((1,), (0,)), ((), ())),
                           preferred_element_type=jnp.float32,
                           precision=_HIGH)
    cnt = jnp.sum(onehot, axis=1, keepdims=True)
    pooled = sums / jnp.maximum(cnt, 1.0)
    hh = jax.nn.relu(_matmul_t(pooled, wl1_ref[...]) + bl1_ref[...])
    out_ref[...] = _matmul_t(hh, wl2_ref[...]) + bl2_ref[...]


_tc_pre = pl.pallas_call(
    _tc_pre_body,
    out_shape=jax.ShapeDtypeStruct((N, HID), jnp.float32),
)

_tc_mid1 = pl.pallas_call(
    _tc_mid1_body,
    out_shape=(jax.ShapeDtypeStruct((N, HID), jnp.float32),
               jax.ShapeDtypeStruct((N, 1), jnp.float32)),
)

_tc_mid2 = pl.pallas_call(
    _tc_mid2_body,
    out_shape=(jax.ShapeDtypeStruct((N, HID), jnp.float32),
               jax.ShapeDtypeStruct((N, HID), jnp.float32)),
)

_tc_final = pl.pallas_call(
    _tc_final_body,
    out_shape=jax.ShapeDtypeStruct((G, 1), jnp.float32),
)


def kernel(x, edge_index, batch, W1, b1, g1, be1, W2, b2, g2, be2,
           Wl1, bl1, Wl2, bl2):
    src = edge_index[0]
    dst = edge_index[1]

    counts = _sc_count(dst)
    h1 = _tc_pre(x, W1)
    hs1, dv = _tc_mid1(counts, h1)
    p1 = _sc_scatter(src, dst, hs1)
    h2, hs2 = _tc_mid2(p1, h1, dv, b1.reshape(1, -1), g1.reshape(1, -1),
                       be1.reshape(1, -1), W2)
    p2 = _sc_scatter(src, dst, hs2)
    out = _tc_final(p2, h2, dv, b2.reshape(1, -1), g2.reshape(1, -1),
                    be2.reshape(1, -1), batch.reshape(1, -1),
                    Wl1, bl1.reshape(1, -1), Wl2, bl2.reshape(1, 1))
    return out.reshape(G)


# speedup vs baseline: 12.9915x; 12.9915x over previous
import functools

import jax
import jax.numpy as jnp
from jax import lax
from jax.experimental import pallas as pl
from jax.experimental.pallas import tpu as pltpu
from jax.experimental.pallas import tpu_sc as plsc

N = 10000
N_PAD = 10112
E = 320000
HID = 128
G = 64

NC = 2
NS = 16
NW = NC * NS
EPW = E // NW
CH = 80
NCHUNK = EPW // CH
RPW = N_PAD // NS
ZR = 8

_MESH = plsc.VectorSubcoreMesh(core_axis_name="c", subcore_axis_name="s")
_HIGH = jax.lax.Precision.HIGHEST


@functools.partial(
    pl.kernel,
    mesh=_MESH,
    out_type=jax.ShapeDtypeStruct((NC, N_PAD, 16), jnp.float32),
    scratch_types=[
        pltpu.VMEM((CH,), jnp.int32),
        pltpu.VMEM((CH, 16), jnp.float32),
        pltpu.VMEM((ZR, 16), jnp.float32),
        pltpu.VMEM_SHARED((N_PAD, 16), jnp.float32),
    ],
)
def _sc_count(dst_hbm, out_hbm, didx, ones_v, zbuf, acc_sh):
    c = lax.axis_index("c")
    s = lax.axis_index("s")

    @pl.loop(0, CH)
    def _(i):
        ones_v[i, :] = jnp.ones((16,), jnp.float32)

    @pl.loop(0, ZR)
    def _(i):
        zbuf[i, :] = jnp.zeros((16,), jnp.float32)

    @pl.loop(0, RPW // ZR)
    def _(i):
        pltpu.sync_copy(zbuf, acc_sh.at[pl.ds(s * RPW + i * ZR, ZR)])

    plsc.subcore_barrier()

    base = (c * NS + s) * EPW

    @pl.loop(0, NCHUNK)
    def _(k):
        pltpu.sync_copy(dst_hbm.at[pl.ds(base + k * CH, CH)], didx)
        pltpu.sync_copy(ones_v, acc_sh.at[didx], add=True)

    plsc.subcore_barrier()
    pltpu.sync_copy(acc_sh.at[pl.ds(s * RPW, RPW)],
                    out_hbm.at[c].at[pl.ds(s * RPW, RPW)])


@functools.partial(
    pl.kernel,
    mesh=_MESH,
    out_type=jax.ShapeDtypeStruct((NC, N_PAD, HID), jnp.float32),
    scratch_types=[
        pltpu.VMEM((CH,), jnp.int32),
        pltpu.VMEM((CH,), jnp.int32),
        pltpu.VMEM((CH, HID), jnp.float32),
        pltpu.VMEM((ZR, HID), jnp.float32),
        pltpu.VMEM_SHARED((N_PAD, HID), jnp.float32),
        pltpu.SemaphoreType.DMA,
    ],
)
def _sc_scatter(src_hbm, dst_hbm, h_hbm, out_hbm, sidx, didx, rows, zbuf,
                acc_sh, sem):
    c = lax.axis_index("c")
    s = lax.axis_index("s")

    @pl.loop(0, ZR)
    def _(i):
        @pl.loop(0, HID // 16)
        def _(j):
            zbuf[i, pl.ds(j * 16, 16)] = jnp.zeros((16,), jnp.float32)

    @pl.loop(0, RPW // ZR)
    def _(i):
        pltpu.sync_copy(zbuf, acc_sh.at[pl.ds(s * RPW + i * ZR, ZR)])

    plsc.subcore_barrier()

    base = (c * NS + s) * EPW

    @pl.loop(0, NCHUNK)
    def _(k):
        pltpu.sync_copy(src_hbm.at[pl.ds(base + k * CH, CH)], sidx)
        pltpu.sync_copy(dst_hbm.at[pl.ds(base + k * CH, CH)], didx)
        pltpu.async_copy(h_hbm.at[sidx], rows, sem).wait()
        pltpu.sync_copy(rows, acc_sh.at[didx], add=True)

    plsc.subcore_barrier()
    pltpu.sync_copy(acc_sh.at[pl.ds(s * RPW, RPW)],
                    out_hbm.at[c].at[pl.ds(s * RPW, RPW)])


def _matmul_t(a, b):
    return lax.dot_general(a, b, (((1,), (1,)), ((), ())),
                           preferred_element_type=jnp.float32,
                           precision=_HIGH)


def _tc_pre_body(x_ref, w1_ref, h1_ref):
    h1_ref[...] = _matmul_t(x_ref[...], w1_ref[...])


def _tc_mid1_body(cnt_ref, h1_ref, hs1_ref, dv_ref):
    deg = cnt_ref[0, 0:N, 0:1] + cnt_ref[1, 0:N, 0:1] + 1.0
    dv = lax.rsqrt(deg)
    dv_ref[...] = dv
    hs1_ref[...] = h1_ref[...] * dv


def _bn_relu(agg, g_ref, be_ref):
    mu = jnp.mean(agg, axis=0, keepdims=True)
    var = jnp.mean((agg - mu) ** 2, axis=0, keepdims=True)
    return jax.nn.relu(g_ref[...] * (agg - mu) * lax.rsqrt(var + 1e-5)
                       + be_ref[...])


def _tc_mid2_body(p_ref, h1_ref, dv_ref, b1_ref, g1_ref, be1_ref, w2_ref,
                  h2_ref, hs2_ref):
    dv = dv_ref[...]
    agg = (p_ref[0, 0:N] + p_ref[1, 0:N]) * dv + dv * dv * h1_ref[...] + b1_ref[...]
    y = _bn_relu(agg, g1_ref, be1_ref)
    h2 = _matmul_t(y, w2_ref[...])
    h2_ref[...] = h2
    hs2_ref[...] = h2 * dv


def _tc_final_body(p_ref, h2_ref, dv_ref, b2_ref, g2_ref, be2_ref, batch_ref,
                   wl1_ref, bl1_ref, wl2_ref, bl2_ref, out_ref):
    dv = dv_ref[...]
    agg = (p_ref[0, 0:N] + p_ref[1, 0:N]) * dv + dv * dv * h2_ref[...] + b2_ref[...]
    y = _bn_relu(agg, g2_ref, be2_ref)
    onehot = (batch_ref[...] == lax.broadcasted_iota(jnp.int32, (G, N), 0)
              ).astype(jnp.float32)
    sums = lax.dot_general(onehot, y, (((1,), (0,)), ((), ())),
                           preferred_element_type=jnp.float32,
                           precision=_HIGH)
    cnt = jnp.sum(onehot, axis=1, keepdims=True)
    pooled = sums / jnp.maximum(cnt, 1.0)
    hh = jax.nn.relu(_matmul_t(pooled, wl1_ref[...]) + bl1_ref[...])
    out_ref[...] = _matmul_t(wl2_ref[...], hh) + bl2_ref[...]


_tc_pre = pl.pallas_call(
    _tc_pre_body,
    out_shape=jax.ShapeDtypeStruct((N, HID), jnp.float32),
)

_tc_mid1 = pl.pallas_call(
    _tc_mid1_body,
    out_shape=(jax.ShapeDtypeStruct((N, HID), jnp.float32),
               jax.ShapeDtypeStruct((N, 1), jnp.float32)),
)

_tc_mid2 = pl.pallas_call(
    _tc_mid2_body,
    out_shape=(jax.ShapeDtypeStruct((N, HID), jnp.float32),
               jax.ShapeDtypeStruct((N, HID), jnp.float32)),
)

_tc_final = pl.pallas_call(
    _tc_final_body,
    out_shape=jax.ShapeDtypeStruct((1, G), jnp.float32),
)


def kernel(x, edge_index, batch, W1, b1, g1, be1, W2, b2, g2, be2,
           Wl1, bl1, Wl2, bl2):
    src = edge_index[0]
    dst = edge_index[1]

    counts = _sc_count(dst)
    h1 = _tc_pre(x, W1)
    hs1, dv = _tc_mid1(counts, h1)
    p1 = _sc_scatter(src, dst, hs1)
    h2, hs2 = _tc_mid2(p1, h1, dv, b1.reshape(1, -1), g1.reshape(1, -1),
                       be1.reshape(1, -1), W2)
    p2 = _sc_scatter(src, dst, hs2)
    out = _tc_final(p2, h2, dv, b2.reshape(1, -1), g2.reshape(1, -1),
                    be2.reshape(1, -1), batch.reshape(1, -1),
                    Wl1, bl1.reshape(1, -1), Wl2,
                    jnp.broadcast_to(bl2.reshape(1, 1), (1, G)))
    return out.reshape(G)


# speedup vs baseline: 31.1152x; 2.3950x over previous
import dataclasses
import functools

import jax
import jax.numpy as jnp
from jax import lax
from jax.experimental import pallas as pl
from jax.experimental.pallas import tpu as pltpu
from jax.experimental.pallas import tpu_sc as plsc

N = 10000
N_PAD = 10112
E = 320000
HID = 128
G = 64

NC = 2
NS = 16
NW = NC * NS
EPW = E // NW
CH = 80
NCHUNK = EPW // CH
RPW = N_PAD // NS
ZR = 8

_MESH = plsc.VectorSubcoreMesh(core_axis_name="c", subcore_axis_name="s")
_HIGH = jax.lax.Precision.HIGHEST

_SC_CP = pltpu.CompilerParams()
if "needs_layout_passes" in pltpu.CompilerParams.__dataclass_fields__:
    _SC_CP = dataclasses.replace(_SC_CP, needs_layout_passes=False)


@functools.partial(
    pl.kernel,
    mesh=_MESH,
    out_type=jax.ShapeDtypeStruct((NW, N_PAD), jnp.float32),
    scratch_types=[
        pltpu.VMEM((EPW,), jnp.int32),
        pltpu.VMEM((N_PAD,), jnp.float32),
    ],
    compiler_params=_SC_CP,
)
def _sc_count(dst_hbm, out_hbm, dstv, acc1):
    c = lax.axis_index("c")
    s = lax.axis_index("s")
    w = c * NS + s

    @pl.loop(0, N_PAD // 16)
    def _(i):
        acc1[pl.ds(i * 16, 16)] = jnp.zeros((16,), jnp.float32)

    pltpu.sync_copy(dst_hbm.at[pl.ds(w * EPW, EPW)], dstv)

    ones16 = jnp.ones((16,), jnp.float32)

    @pl.loop(0, EPW // 16)
    def _(i):
        idx = dstv[pl.ds(i * 16, 16)]
        plsc.addupdate_scatter(acc1, [idx], ones16)

    pltpu.sync_copy(acc1, out_hbm.at[w])


@functools.partial(
    pl.kernel,
    mesh=_MESH,
    out_type=jax.ShapeDtypeStruct((NC, N_PAD, HID), jnp.float32),
    scratch_types=[
        pltpu.VMEM((EPW,), jnp.int32),
        pltpu.VMEM((CH,), jnp.int32),
        pltpu.VMEM((CH,), jnp.int32),
        pltpu.VMEM((CH, HID), jnp.float32),
        pltpu.VMEM((CH, HID), jnp.float32),
        pltpu.VMEM((ZR, HID), jnp.float32),
        pltpu.VMEM_SHARED((N_PAD, HID), jnp.float32),
        pltpu.SemaphoreType.DMA,
        pltpu.SemaphoreType.DMA,
        pltpu.SemaphoreType.DMA,
        pltpu.SemaphoreType.DMA,
    ],
)
def _sc_scatter(src_hbm, dst_hbm, h_hbm, out_hbm, sall, d0, d1, r0, r1, zbuf,
                acc_sh, gs0, gs1, ds0, ds1):
    c = lax.axis_index("c")
    s = lax.axis_index("s")

    @pl.loop(0, ZR)
    def _(i):
        @pl.loop(0, HID // 16)
        def _(j):
            zbuf[i, pl.ds(j * 16, 16)] = jnp.zeros((16,), jnp.float32)

    @pl.loop(0, RPW // ZR)
    def _(i):
        pltpu.sync_copy(zbuf, acc_sh.at[pl.ds(s * RPW + i * ZR, ZR)])

    plsc.subcore_barrier()

    base = (c * NS + s) * EPW
    pltpu.sync_copy(src_hbm.at[pl.ds(base, EPW)], sall)

    didx = (d0, d1)
    rows = (r0, r1)
    gsem = (gs0, gs1)
    dsem = (ds0, ds1)

    def _didx_copy(kb, b):
        return pltpu.make_async_copy(
            dst_hbm.at[pl.ds(base + kb * CH, CH)], didx[b], dsem[b])

    def _gather_copy(kb, b):
        return pltpu.make_async_copy(
            h_hbm.at[sall.at[pl.ds(kb * CH, CH)]], rows[b], gsem[b])

    def _start(kb, b):
        _didx_copy(kb, b).start()
        _gather_copy(kb, b).start()

    def _finish(kb, b):
        _gather_copy(kb, b).wait()
        _didx_copy(kb, b).wait()
        pltpu.sync_copy(rows[b], acc_sh.at[didx[b]], add=True)

    _start(0, 0)

    @pl.loop(0, NCHUNK - 1, step=2)
    def _(k):
        for b in range(2):
            kb = k + b
            _start(kb + 1, 1 - b)
            _finish(kb, b)

    _finish(NCHUNK - 1, 0)

    plsc.subcore_barrier()
    pltpu.sync_copy(acc_sh.at[pl.ds(s * RPW, RPW)],
                    out_hbm.at[c].at[pl.ds(s * RPW, RPW)])


def _matmul_t(a, b):
    return lax.dot_general(a, b, (((1,), (1,)), ((), ())),
                           preferred_element_type=jnp.float32,
                           precision=_HIGH)


def _tc_pre_body(x_ref, w1_ref, h1_ref):
    h1_ref[...] = _matmul_t(x_ref[...], w1_ref[...])


def _tc_mid1_body(cnt_ref, h1_ref, hs1_ref, dv_ref):
    deg = lax.dot_general(cnt_ref[...], jnp.ones((NW, 1), jnp.float32),
                          (((0,), (0,)), ((), ())),
                          preferred_element_type=jnp.float32,
                          precision=_HIGH)[0:N] + 1.0
    dv = lax.rsqrt(deg)
    dv_ref[...] = dv
    hs1_ref[...] = h1_ref[...] * dv


def _bn_relu(agg, g_ref, be_ref):
    mu = jnp.mean(agg, axis=0, keepdims=True)
    var = jnp.mean((agg - mu) ** 2, axis=0, keepdims=True)
    return jax.nn.relu(g_ref[...] * (agg - mu) * lax.rsqrt(var + 1e-5)
                       + be_ref[...])


def _tc_mid2_body(p_ref, h1_ref, dv_ref, b1_ref, g1_ref, be1_ref, w2_ref,
                  h2_ref, hs2_ref):
    dv = dv_ref[...]
    agg = (p_ref[0, 0:N] + p_ref[1, 0:N]) * dv + dv * dv * h1_ref[...] + b1_ref[...]
    y = _bn_relu(agg, g1_ref, be1_ref)
    h2 = _matmul_t(y, w2_ref[...])
    h2_ref[...] = h2
    hs2_ref[...] = h2 * dv


def _tc_final_body(p_ref, h2_ref, dv_ref, b2_ref, g2_ref, be2_ref, batch_ref,
                   wl1_ref, bl1_ref, wl2_ref, bl2_ref, out_ref):
    dv = dv_ref[...]
    agg = (p_ref[0, 0:N] + p_ref[1, 0:N]) * dv + dv * dv * h2_ref[...] + b2_ref[...]
    y = _bn_relu(agg, g2_ref, be2_ref)
    onehot = (batch_ref[...] == lax.broadcasted_iota(jnp.int32, (G, N), 0)
              ).astype(jnp.float32)
    sums = lax.dot_general(onehot, y, (((1,), (0,)), ((), ())),
                           preferred_element_type=jnp.float32,
                           precision=_HIGH)
    cnt = jnp.sum(onehot, axis=1, keepdims=True)
    pooled = sums / jnp.maximum(cnt, 1.0)
    hh = jax.nn.relu(_matmul_t(pooled, wl1_ref[...]) + bl1_ref[...])
    out_ref[...] = _matmul_t(wl2_ref[...], hh) + bl2_ref[...]


_tc_pre = pl.pallas_call(
    _tc_pre_body,
    out_shape=jax.ShapeDtypeStruct((N, HID), jnp.float32),
)

_tc_mid1 = pl.pallas_call(
    _tc_mid1_body,
    out_shape=(jax.ShapeDtypeStruct((N, HID), jnp.float32),
               jax.ShapeDtypeStruct((N, 1), jnp.float32)),
)

_tc_mid2 = pl.pallas_call(
    _tc_mid2_body,
    out_shape=(jax.ShapeDtypeStruct((N, HID), jnp.float32),
               jax.ShapeDtypeStruct((N, HID), jnp.float32)),
)

_tc_final = pl.pallas_call(
    _tc_final_body,
    out_shape=jax.ShapeDtypeStruct((1, G), jnp.float32),
)


def kernel(x, edge_index, batch, W1, b1, g1, be1, W2, b2, g2, be2,
           Wl1, bl1, Wl2, bl2):
    src = edge_index[0]
    dst = edge_index[1]

    counts = _sc_count(dst)
    h1 = _tc_pre(x, W1)
    hs1, dv = _tc_mid1(counts, h1)
    p1 = _sc_scatter(src, dst, hs1)
    h2, hs2 = _tc_mid2(p1, h1, dv, b1.reshape(1, -1), g1.reshape(1, -1),
                       be1.reshape(1, -1), W2)
    p2 = _sc_scatter(src, dst, hs2)
    out = _tc_final(p2, h2, dv, b2.reshape(1, -1), g2.reshape(1, -1),
                    be2.reshape(1, -1), batch.reshape(1, -1),
                    Wl1, bl1.reshape(1, -1), Wl2,
                    jnp.broadcast_to(bl2.reshape(1, 1), (1, G)))
    return out.reshape(G)


# speedup vs baseline: 36.5976x; 1.1762x over previous
import dataclasses
import functools

import jax
import jax.numpy as jnp
from jax import lax
from jax.experimental import pallas as pl
from jax.experimental.pallas import tpu as pltpu
from jax.experimental.pallas import tpu_sc as plsc

N = 10000
N_PAD = 10112
E = 320000
HID = 128
G = 64

NC = 2
NS = 16
NW = NC * NS
EPW = E // NW
CH = 80
NCHUNK = EPW // CH
RPW = N_PAD // NS
ZR = 8

_MESH = plsc.VectorSubcoreMesh(core_axis_name="c", subcore_axis_name="s")
_HIGH = jax.lax.Precision.HIGHEST

_SC_CP = pltpu.CompilerParams()
if "needs_layout_passes" in pltpu.CompilerParams.__dataclass_fields__:
    _SC_CP = dataclasses.replace(_SC_CP, needs_layout_passes=False)


@functools.partial(
    pl.kernel,
    mesh=_MESH,
    out_type=jax.ShapeDtypeStruct((NW, N_PAD), jnp.float32),
    scratch_types=[
        pltpu.VMEM((EPW,), jnp.int32),
        pltpu.VMEM((N_PAD,), jnp.float32),
    ],
    compiler_params=_SC_CP,
)
def _sc_count(dst_hbm, out_hbm, dstv, acc1):
    c = lax.axis_index("c")
    s = lax.axis_index("s")
    w = c * NS + s

    @pl.loop(0, N_PAD // 16)
    def _(i):
        acc1[pl.ds(i * 16, 16)] = jnp.zeros((16,), jnp.float32)

    pltpu.sync_copy(dst_hbm.at[pl.ds(w * EPW, EPW)], dstv)

    ones16 = jnp.ones((16,), jnp.float32)

    @pl.loop(0, EPW // 16)
    def _(i):
        idx = dstv[pl.ds(i * 16, 16)]
        plsc.addupdate_scatter(acc1, [idx], ones16)

    pltpu.sync_copy(acc1, out_hbm.at[w])


@functools.partial(
    pl.kernel,
    mesh=_MESH,
    out_type=jax.ShapeDtypeStruct((NC, N_PAD, HID), jnp.float32),
    scratch_types=[
        pltpu.VMEM((EPW,), jnp.int32),
        pltpu.VMEM((CH,), jnp.int32),
        pltpu.VMEM((CH,), jnp.int32),
        pltpu.VMEM((CH,), jnp.int32),
        pltpu.VMEM((CH, HID), jnp.float32),
        pltpu.VMEM((CH, HID), jnp.float32),
        pltpu.VMEM((CH, HID), jnp.float32),
        pltpu.VMEM((ZR, HID), jnp.float32),
        pltpu.VMEM_SHARED((N_PAD, HID), jnp.float32),
        pltpu.SemaphoreType.DMA,
        pltpu.SemaphoreType.DMA,
        pltpu.SemaphoreType.DMA,
        pltpu.SemaphoreType.DMA,
        pltpu.SemaphoreType.DMA,
        pltpu.SemaphoreType.DMA,
        pltpu.SemaphoreType.DMA,
    ],
)
def _sc_scatter(src_hbm, dst_hbm, h_hbm, out_hbm, sall, d0, d1, d2,
                r0, r1, r2, zbuf, acc_sh,
                gs0, gs1, gs2, ds0, ds1, ds2, zsem):
    c = lax.axis_index("c")
    s = lax.axis_index("s")

    @pl.loop(0, ZR)
    def _(i):
        @pl.loop(0, HID // 16)
        def _(j):
            zbuf[i, pl.ds(j * 16, 16)] = jnp.zeros((16,), jnp.float32)

    @pl.loop(0, RPW // ZR)
    def _(i):
        pltpu.make_async_copy(
            zbuf, acc_sh.at[pl.ds(s * RPW + i * ZR, ZR)], zsem).start()

    base = (c * NS + s) * EPW
    pltpu.sync_copy(src_hbm.at[pl.ds(base, EPW)], sall)

    didx = (d0, d1, d2)
    rows = (r0, r1, r2)
    gsem = (gs0, gs1, gs2)
    dsem = (ds0, ds1, ds2)

    def _didx_copy(kb, b):
        return pltpu.make_async_copy(
            dst_hbm.at[pl.ds(base + kb * CH, CH)], didx[b], dsem[b])

    def _gather_copy(kb, b):
        return pltpu.make_async_copy(
            h_hbm.at[sall.at[pl.ds(kb * CH, CH)]], rows[b], gsem[b])

    def _start(kb, b):
        _didx_copy(kb, b).start()
        _gather_copy(kb, b).start()

    def _finish(kb, b):
        _gather_copy(kb, b).wait()
        _didx_copy(kb, b).wait()
        pltpu.sync_copy(rows[b], acc_sh.at[didx[b]], add=True)

    _start(0, 0)
    _start(1, 1)

    @pl.loop(0, RPW // ZR)
    def _(i):
        pltpu.make_async_copy(
            zbuf, acc_sh.at[pl.ds(s * RPW + i * ZR, ZR)], zsem).wait()

    plsc.subcore_barrier()

    @pl.loop(0, NCHUNK - 2, step=3)
    def _(k):
        for b in range(3):
            kb = k + b
            _start(kb + 2, (b + 2) % 3)
            _finish(kb, b)

    _finish(NCHUNK - 2, (NCHUNK - 2) % 3)
    _finish(NCHUNK - 1, (NCHUNK - 1) % 3)

    plsc.subcore_barrier()
    pltpu.sync_copy(acc_sh.at[pl.ds(s * RPW, RPW)],
                    out_hbm.at[c].at[pl.ds(s * RPW, RPW)])


def _matmul_t(a, b):
    return lax.dot_general(a, b, (((1,), (1,)), ((), ())),
                           preferred_element_type=jnp.float32,
                           precision=_HIGH)


def _tc_mid1_body(cnt_ref, x_ref, w1_ref, h1_ref, hs1_ref, dv_ref):
    deg = lax.dot_general(cnt_ref[...], jnp.ones((NW, 1), jnp.float32),
                          (((0,), (0,)), ((), ())),
                          preferred_element_type=jnp.float32,
                          precision=_HIGH)[0:N] + 1.0
    dv = lax.rsqrt(deg)
    dv_ref[...] = dv
    h1 = _matmul_t(x_ref[...], w1_ref[...])
    h1_ref[...] = h1
    hs1_ref[...] = h1 * dv


def _bn_relu(agg, g_ref, be_ref):
    mu = jnp.mean(agg, axis=0, keepdims=True)
    var = jnp.mean((agg - mu) ** 2, axis=0, keepdims=True)
    return jax.nn.relu(g_ref[...] * (agg - mu) * lax.rsqrt(var + 1e-5)
                       + be_ref[...])


def _tc_mid2_body(p_ref, h1_ref, dv_ref, b1_ref, g1_ref, be1_ref, w2_ref,
                  h2_ref, hs2_ref):
    dv = dv_ref[...]
    agg = (p_ref[0, 0:N] + p_ref[1, 0:N]) * dv + dv * dv * h1_ref[...] + b1_ref[...]
    y = _bn_relu(agg, g1_ref, be1_ref)
    h2 = _matmul_t(y, w2_ref[...])
    h2_ref[...] = h2
    hs2_ref[...] = h2 * dv


def _tc_final_body(p_ref, h2_ref, dv_ref, b2_ref, g2_ref, be2_ref, batch_ref,
                   wl1_ref, bl1_ref, wl2_ref, bl2_ref, out_ref):
    dv = dv_ref[...]
    agg = (p_ref[0, 0:N] + p_ref[1, 0:N]) * dv + dv * dv * h2_ref[...] + b2_ref[...]
    y = _bn_relu(agg, g2_ref, be2_ref)
    onehot = (batch_ref[...] == lax.broadcasted_iota(jnp.int32, (G, N), 0)
              ).astype(jnp.float32)
    sums = lax.dot_general(onehot, y, (((1,), (0,)), ((), ())),
                           preferred_element_type=jnp.float32,
                           precision=_HIGH)
    cnt = jnp.sum(onehot, axis=1, keepdims=True)
    pooled = sums / jnp.maximum(cnt, 1.0)
    hh = jax.nn.relu(_matmul_t(pooled, wl1_ref[...]) + bl1_ref[...])
    out_ref[...] = _matmul_t(wl2_ref[...], hh) + bl2_ref[...]


_tc_mid1 = pl.pallas_call(
    _tc_mid1_body,
    out_shape=(jax.ShapeDtypeStruct((N, HID), jnp.float32),
               jax.ShapeDtypeStruct((N, HID), jnp.float32),
               jax.ShapeDtypeStruct((N, 1), jnp.float32)),
)

_tc_mid2 = pl.pallas_call(
    _tc_mid2_body,
    out_shape=(jax.ShapeDtypeStruct((N, HID), jnp.float32),
               jax.ShapeDtypeStruct((N, HID), jnp.float32)),
)

_tc_final = pl.pallas_call(
    _tc_final_body,
    out_shape=jax.ShapeDtypeStruct((1, G), jnp.float32),
)


def kernel(x, edge_index, batch, W1, b1, g1, be1, W2, b2, g2, be2,
           Wl1, bl1, Wl2, bl2):
    src = edge_index[0]
    dst = edge_index[1]

    counts = _sc_count(dst)
    h1, hs1, dv = _tc_mid1(counts, x, W1)
    p1 = _sc_scatter(src, dst, hs1)
    h2, hs2 = _tc_mid2(p1, h1, dv, b1.reshape(1, -1), g1.reshape(1, -1),
                       be1.reshape(1, -1), W2)
    p2 = _sc_scatter(src, dst, hs2)
    out = _tc_final(p2, h2, dv, b2.reshape(1, -1), g2.reshape(1, -1),
                    be2.reshape(1, -1), batch.reshape(1, -1),
                    Wl1, bl1.reshape(1, -1), Wl2,
                    jnp.broadcast_to(bl2.reshape(1, 1), (1, G)))
    return out.reshape(G)
